# baseline (speedup 1.0000x reference)
import jax
import jax.numpy as jnp
from jax.experimental import pallas as pl
from jax.experimental.pallas import tpu as pltpu
from jax.experimental.pallas import tpu_sc as plsc

D = 1024
L = 12
FF = 2816
EPS = 1e-5

ATT_T = 1024
NA = D // ATT_T
FF_T = 1408
NF = FF // FF_T
NJ = NA + NF

V_T = 4096


def _rms(x, w):
    return x * jax.lax.rsqrt(jnp.mean(x * x, axis=-1, keepdims=True) + EPS) * w


def _sc_gather(table, idx2d):
    n = idx2d.shape[1]
    mesh = plsc.VectorSubcoreMesh(core_axis_name="c", subcore_axis_name="s")

    @pl.kernel(out_type=jax.ShapeDtypeStruct((n, table.shape[1]), table.dtype),
               mesh=mesh)
    def gk(tab_hbm, i_hbm, o_hbm):
        def body(i_vmem, o_vmem):
            pltpu.sync_copy(tab_hbm.at[i_vmem.at[0]], o_vmem)

        pltpu.emit_pipeline(
            body,
            grid=(1,),
            in_specs=[pl.BlockSpec((1, n), lambda i: (0, 0))],
            out_specs=[pl.BlockSpec((n, table.shape[1]), lambda i: (0, 0))],
            core_axis_name="s",
            dimension_semantics=(pltpu.PARALLEL,),
        )(i_hbm, o_hbm)

    return gk(table, idx2d)


def _layers_body(x0_ref, Wv_ref, Wo_ref, W1_ref, W2_ref, W3_ref,
                 ln1_ref, ln2_ref, lno_ref, out_ref, x_s, h_s):
    i = pl.program_id(0)
    j = pl.program_id(1)

    @pl.when(jnp.logical_and(i == 0, j == 0))
    def _():
        x_s[...] = x0_ref[...]

    @pl.when(j == 0)
    def _():
        h_s[...] = _rms(x_s[...], ln1_ref[0])

    @pl.when(j == NA)
    def _():
        h_s[...] = _rms(x_s[...], ln2_ref[0])

    h = h_s[...]

    @pl.when(j < NA)
    def _():
        t = jnp.dot(h, Wv_ref[0], preferred_element_type=jnp.float32)
        x_s[...] = x_s[...] + jnp.dot(t, Wo_ref[0],
                                      preferred_element_type=jnp.float32)

    @pl.when(j >= NA)
    def _():
        a = jnp.dot(h, W1_ref[0], preferred_element_type=jnp.float32)
        b = jnp.dot(h, W2_ref[0], preferred_element_type=jnp.float32)
        g = (a * jax.lax.logistic(a)) * b
        x_s[...] = x_s[...] + jnp.dot(g, W3_ref[0],
                                      preferred_element_type=jnp.float32)

    @pl.when(jnp.logical_and(i == L - 1, j == NJ - 1))
    def _():
        out_ref[...] = _rms(x_s[...], lno_ref[...])


def _run_layers(x0, Wv, Wo, W1, W2, W3, ln1, ln2, lno):
    B = x0.shape[0]
    return pl.pallas_call(
        _layers_body,
        grid=(L, NJ),
        in_specs=[
            pl.BlockSpec((B, D), lambda i, j: (0, 0)),
            pl.BlockSpec((1, D, ATT_T),
                         lambda i, j: (i, 0, jnp.minimum(j, NA - 1))),
            pl.BlockSpec((1, ATT_T, D),
                         lambda i, j: (i, jnp.minimum(j, NA - 1), 0)),
            pl.BlockSpec((1, D, FF_T),
                         lambda i, j: (i, 0, jnp.clip(j - NA, 0, NF - 1))),
            pl.BlockSpec((1, D, FF_T),
                         lambda i, j: (i, 0, jnp.clip(j - NA, 0, NF - 1))),
            pl.BlockSpec((1, FF_T, D),
                         lambda i, j: (i, jnp.clip(j - NA, 0, NF - 1), 0)),
            pl.BlockSpec((1, 1, D), lambda i, j: (i, 0, 0)),
            pl.BlockSpec((1, 1, D), lambda i, j: (i, 0, 0)),
            pl.BlockSpec((1, D), lambda i, j: (0, 0)),
        ],
        out_specs=pl.BlockSpec((B, D), lambda i, j: (0, 0)),
        out_shape=jax.ShapeDtypeStruct((B, D), jnp.float32),
        scratch_shapes=[pltpu.VMEM((B, D), jnp.float32),
                        pltpu.VMEM((B, D), jnp.float32)],
    )(x0, Wv, Wo, W1, W2, W3,
      ln1.reshape(L, 1, D), ln2.reshape(L, 1, D), lno)


def _head_body(x_ref, tab_ref, out_ref):
    out_ref[...] = jax.lax.dot_general(
        x_ref[...], tab_ref[...], (((1,), (1,)), ((), ())),
        preferred_element_type=jnp.float32)[:, None, :]


def _run_head(xn, table):
    B = xn.shape[0]
    V = table.shape[0]
    return pl.pallas_call(
        _head_body,
        grid=(pl.cdiv(V, V_T),),
        in_specs=[
            pl.BlockSpec((B, D), lambda v: (0, 0)),
            pl.BlockSpec((V_T, D), lambda v: (v, 0)),
        ],
        out_specs=pl.BlockSpec((B, 1, V_T), lambda v: (0, 0, v)),
        out_shape=jax.ShapeDtypeStruct((B, 1, V), jnp.float32),
    )(xn, table)


def kernel(table, Wq, Wk, Wv, Wo, W1, W2, W3, ln1, ln2, ln_out, tokens):
    B, T = tokens.shape
    assert T == 1, "kernel exploits T == 1 (single-position decode)"
    V = table.shape[0]
    idx = tokens.reshape(1, B * T).astype(jnp.int32)
    x0 = _sc_gather(table, idx)
    xn = _run_layers(x0, Wv, Wo, W1, W2, W3, ln1, ln2, ln_out.reshape(1, D))
    return _run_head(xn, table)


# speedup vs baseline: 1.0108x; 1.0108x over previous
import jax
import jax.numpy as jnp
from jax.experimental import pallas as pl
from jax.experimental.pallas import tpu as pltpu
from jax.experimental.pallas import tpu_sc as plsc

D = 1024
L = 12
FF = 2816
EPS = 1e-5

ATT_T = 512
NA = D // ATT_T
FF_T = 1408
NF = FF // FF_T
NJ = NA + NF
S_HEAD = L * NJ

V_T = 2048


def _rms(x, w):
    return x * jax.lax.rsqrt(jnp.mean(x * x, axis=-1, keepdims=True) + EPS) * w


def _sc_gather(table, idx2d):
    n = idx2d.shape[1]
    mesh = plsc.VectorSubcoreMesh(core_axis_name="c", subcore_axis_name="s")

    @pl.kernel(out_type=jax.ShapeDtypeStruct((n, table.shape[1]), table.dtype),
               mesh=mesh)
    def gk(tab_hbm, i_hbm, o_hbm):
        def body(i_vmem, o_vmem):
            pltpu.sync_copy(tab_hbm.at[i_vmem.at[0]], o_vmem)

        pltpu.emit_pipeline(
            body,
            grid=(1,),
            in_specs=[pl.BlockSpec((1, n), lambda i: (0, 0))],
            out_specs=[pl.BlockSpec((n, table.shape[1]), lambda i: (0, 0))],
            core_axis_name="s",
            dimension_semantics=(pltpu.PARALLEL,),
        )(i_hbm, o_hbm)

    return gk(table, idx2d)


def _li(s):
    return jnp.minimum(s // NJ, L - 1)


def _lj(s):
    return jnp.where(s < S_HEAD, s % NJ, NJ - 1)


def _vt(s, nv):
    return jnp.clip(s - S_HEAD, 0, nv - 1)


def _fused_body(x0_ref, Wv_ref, Wo_ref, W1_ref, W2_ref, W3_ref,
                ln1_ref, ln2_ref, lno_ref, tab_ref, out_ref, x_s, h_s):
    s = pl.program_id(0)
    lj = s % NJ
    in_layers = s < S_HEAD

    @pl.when(s == 0)
    def _():
        x_s[...] = x0_ref[...]

    @pl.when(jnp.logical_and(in_layers, lj == 0))
    def _():
        h_s[...] = _rms(x_s[...], ln1_ref[0])

    @pl.when(jnp.logical_and(in_layers, lj == NA))
    def _():
        h_s[...] = _rms(x_s[...], ln2_ref[0])

    h = h_s[...]

    @pl.when(jnp.logical_and(in_layers, lj < NA))
    def _():
        t = jnp.dot(h, Wv_ref[0], preferred_element_type=jnp.float32)
        x_s[...] = x_s[...] + jnp.dot(t, Wo_ref[0],
                                      preferred_element_type=jnp.float32)

    @pl.when(jnp.logical_and(in_layers, lj >= NA))
    def _():
        a = jnp.dot(h, W1_ref[0], preferred_element_type=jnp.float32)
        b = jnp.dot(h, W2_ref[0], preferred_element_type=jnp.float32)
        g = (a * jax.lax.logistic(a)) * b
        x_s[...] = x_s[...] + jnp.dot(g, W3_ref[0],
                                      preferred_element_type=jnp.float32)

    @pl.when(s == S_HEAD)
    def _():
        h_s[...] = _rms(x_s[...], lno_ref[...])

    @pl.when(jnp.logical_not(in_layers))
    def _():
        out_ref[...] = jax.lax.dot_general(
            h_s[...], tab_ref[...], (((1,), (1,)), ((), ())),
            preferred_element_type=jnp.float32)[:, None, :]


def _run_fused(x0, Wv, Wo, W1, W2, W3, ln1, ln2, lno, table):
    B = x0.shape[0]
    V = table.shape[0]
    nv = pl.cdiv(V, V_T)
    return pl.pallas_call(
        _fused_body,
        grid=(S_HEAD + nv,),
        in_specs=[
            pl.BlockSpec((B, D), lambda s: (0, 0)),
            pl.BlockSpec((1, D, ATT_T),
                         lambda s: (_li(s), 0, jnp.minimum(_lj(s), NA - 1))),
            pl.BlockSpec((1, ATT_T, D),
                         lambda s: (_li(s), jnp.minimum(_lj(s), NA - 1), 0)),
            pl.BlockSpec((1, D, FF_T),
                         lambda s: (_li(s), 0, jnp.clip(_lj(s) - NA, 0, NF - 1))),
            pl.BlockSpec((1, D, FF_T),
                         lambda s: (_li(s), 0, jnp.clip(_lj(s) - NA, 0, NF - 1))),
            pl.BlockSpec((1, FF_T, D),
                         lambda s: (_li(s), jnp.clip(_lj(s) - NA, 0, NF - 1), 0)),
            pl.BlockSpec((1, 1, D), lambda s: (_li(s), 0, 0)),
            pl.BlockSpec((1, 1, D), lambda s: (_li(s), 0, 0)),
            pl.BlockSpec((1, D), lambda s: (0, 0)),
            pl.BlockSpec((V_T, D), lambda s: (_vt(s, nv), 0)),
        ],
        out_specs=pl.BlockSpec((B, 1, V_T), lambda s: (0, 0, _vt(s, nv))),
        out_shape=jax.ShapeDtypeStruct((B, 1, V), jnp.float32),
        scratch_shapes=[pltpu.VMEM((B, D), jnp.float32),
                        pltpu.VMEM((B, D), jnp.float32)],
    )(x0, Wv, Wo, W1, W2, W3,
      ln1.reshape(L, 1, D), ln2.reshape(L, 1, D), lno, table)


def kernel(table, Wq, Wk, Wv, Wo, W1, W2, W3, ln1, ln2, ln_out, tokens):
    B, T = tokens.shape
    assert T == 1, "kernel exploits T == 1 (single-position decode)"
    idx = tokens.reshape(1, B * T).astype(jnp.int32)
    x0 = _sc_gather(table, idx)
    return _run_fused(x0, Wv, Wo, W1, W2, W3, ln1, ln2,
                      ln_out.reshape(1, D), table)


# speedup vs baseline: 1.0149x; 1.0040x over previous
import jax
import jax.numpy as jnp
from jax.experimental import pallas as pl
from jax.experimental.pallas import tpu as pltpu
from jax.experimental.pallas import tpu_sc as plsc

D = 1024
L = 12
FF = 2816
EPS = 1e-5

ATT_T = 512
NA = D // ATT_T
FF_T = 1408
NF = FF // FF_T
NJ = NA + NF
S_HEAD = L * NJ

V_T = 2048


def _rms(x, w):
    return x * jax.lax.rsqrt(jnp.mean(x * x, axis=-1, keepdims=True) + EPS) * w


def _sc_gather(table, idx2d):
    n = idx2d.shape[1]
    mesh = plsc.VectorSubcoreMesh(core_axis_name="c", subcore_axis_name="s")

    @pl.kernel(out_type=jax.ShapeDtypeStruct((n, table.shape[1]), table.dtype),
               mesh=mesh)
    def gk(tab_hbm, i_hbm, o_hbm):
        def body(i_vmem, o_vmem):
            pltpu.sync_copy(tab_hbm.at[i_vmem.at[0]], o_vmem)

        pltpu.emit_pipeline(
            body,
            grid=(1,),
            in_specs=[pl.BlockSpec((1, n), lambda i: (0, 0))],
            out_specs=[pl.BlockSpec((n, table.shape[1]), lambda i: (0, 0))],
            core_axis_name="s",
            dimension_semantics=(pltpu.PARALLEL,),
        )(i_hbm, o_hbm)

    return gk(table, idx2d)


def _li(s):
    return jnp.minimum(s // NJ, L - 1)


def _lj(s):
    return jnp.where(s < S_HEAD, s % NJ, NJ - 1)


def _vt(s, nv):
    return jnp.clip(s - S_HEAD, 0, nv - 1)


def _lw(s):
    li, lj = _li(s), _lj(s)
    return jnp.where(lj >= NA, li, jnp.maximum(li - 1, 0))


def _tw(s):
    li, lj = _li(s), _lj(s)
    return jnp.where(lj >= NA, lj - NA,
                     jnp.where(li > 0, NF - 1, 0))


def _fused_body(x0_ref, Wv_ref, Wo_ref, W1_ref, W2_ref, W3_ref,
                ln1_ref, ln2_ref, lno_ref, tab_ref, out_ref, x_s, h_s):
    s = pl.program_id(0)
    lj = s % NJ
    in_layers = s < S_HEAD

    @pl.when(s == 0)
    def _():
        x_s[...] = x0_ref[...]

    @pl.when(jnp.logical_and(in_layers, lj == 0))
    def _():
        h_s[...] = _rms(x_s[...], ln1_ref[0])

    @pl.when(jnp.logical_and(in_layers, lj == NA))
    def _():
        h_s[...] = _rms(x_s[...], ln2_ref[0])

    h = h_s[...]

    @pl.when(jnp.logical_and(in_layers, lj < NA))
    def _():
        t = jnp.dot(h, Wv_ref[0], preferred_element_type=jnp.float32)
        x_s[...] = x_s[...] + jnp.dot(t, Wo_ref[0],
                                      preferred_element_type=jnp.float32)

    @pl.when(jnp.logical_and(in_layers, lj >= NA))
    def _():
        a = jnp.dot(h, W1_ref[0], preferred_element_type=jnp.float32)
        b = jnp.dot(h, W2_ref[0], preferred_element_type=jnp.float32)
        g = (a * jax.lax.logistic(a)) * b
        x_s[...] = x_s[...] + jnp.dot(g, W3_ref[0],
                                      preferred_element_type=jnp.float32)

    @pl.when(s == S_HEAD)
    def _():
        h_s[...] = _rms(x_s[...], lno_ref[...])

    @pl.when(jnp.logical_not(in_layers))
    def _():
        out_ref[...] = jax.lax.dot_general(
            h_s[...], tab_ref[...], (((1,), (1,)), ((), ())),
            preferred_element_type=jnp.float32)[:, None, :]


def _run_fused(x0, Wv, Wo, W1, W2, W3, ln1, ln2, lno, table):
    B = x0.shape[0]
    V = table.shape[0]
    nv = pl.cdiv(V, V_T)
    return pl.pallas_call(
        _fused_body,
        grid=(S_HEAD + nv,),
        in_specs=[
            pl.BlockSpec((B, D), lambda s: (0, 0)),
            pl.BlockSpec((1, D, ATT_T),
                         lambda s: (_li(s), 0, jnp.minimum(_lj(s), NA - 1))),
            pl.BlockSpec((1, ATT_T, D),
                         lambda s: (_li(s), jnp.minimum(_lj(s), NA - 1), 0)),
            pl.BlockSpec((1, D, FF_T),
                         lambda s: (_lw(s), 0, _tw(s))),
            pl.BlockSpec((1, D, FF_T),
                         lambda s: (_lw(s), 0, _tw(s))),
            pl.BlockSpec((1, FF_T, D),
                         lambda s: (_lw(s), _tw(s), 0)),
            pl.BlockSpec((1, 1, D), lambda s: (_li(s), 0, 0)),
            pl.BlockSpec((1, 1, D), lambda s: (_li(s), 0, 0)),
            pl.BlockSpec((1, D), lambda s: (0, 0)),
            pl.BlockSpec((V_T, D), lambda s: (_vt(s, nv), 0)),
        ],
        out_specs=pl.BlockSpec((B, 1, V_T), lambda s: (0, 0, _vt(s, nv))),
        out_shape=jax.ShapeDtypeStruct((B, 1, V), jnp.float32),
        scratch_shapes=[pltpu.VMEM((B, D), jnp.float32),
                        pltpu.VMEM((B, D), jnp.float32)],
    )(x0, Wv, Wo, W1, W2, W3,
      ln1.reshape(L, 1, D), ln2.reshape(L, 1, D), lno, table)


def kernel(table, Wq, Wk, Wv, Wo, W1, W2, W3, ln1, ln2, ln_out, tokens):
    B, T = tokens.shape
    assert T == 1, "kernel exploits T == 1 (single-position decode)"
    idx = tokens.reshape(1, B * T).astype(jnp.int32)
    x0 = _sc_gather(table, idx)
    return _run_fused(x0, Wv, Wo, W1, W2, W3, ln1, ln2,
                      ln_out.reshape(1, D), table)


# speedup vs baseline: 1.0151x; 1.0002x over previous
import jax
import jax.numpy as jnp
from jax.experimental import pallas as pl
from jax.experimental.pallas import tpu as pltpu
from jax.experimental.pallas import tpu_sc as plsc

D = 1024
L = 12
FF = 2816
EPS = 1e-5

ATT_T = 512
NA = D // ATT_T
FF_T = 1408
NF = FF // FF_T
NJ = NA + NF
S_HEAD = L * NJ

V_T = 2048


def _rms(x, w):
    return x * jax.lax.rsqrt(jnp.mean(x * x, axis=-1, keepdims=True) + EPS) * w


def _sc_gather(table, idx2d):
    n = idx2d.shape[1]
    mesh = plsc.VectorSubcoreMesh(core_axis_name="c", subcore_axis_name="s")

    @pl.kernel(out_type=jax.ShapeDtypeStruct((n, table.shape[1]), table.dtype),
               mesh=mesh)
    def gk(tab_hbm, i_hbm, o_hbm):
        def body(i_vmem, o_vmem):
            pltpu.sync_copy(tab_hbm.at[i_vmem.at[0]], o_vmem)

        pltpu.emit_pipeline(
            body,
            grid=(1,),
            in_specs=[pl.BlockSpec((1, n), lambda i: (0, 0))],
            out_specs=[pl.BlockSpec((n, table.shape[1]), lambda i: (0, 0))],
            core_axis_name="s",
            dimension_semantics=(pltpu.PARALLEL,),
        )(i_hbm, o_hbm)

    return gk(table, idx2d)


def _li(s):
    return jnp.minimum(s // NJ, L - 1)


def _lj(s):
    return jnp.where(s < S_HEAD, s % NJ, NJ - 1)


def _vt(s, nv):
    return jnp.clip(s - S_HEAD, 0, nv - 1)


def _lw(s):
    li, lj = _li(s), _lj(s)
    return jnp.where(lj >= NA, li, jnp.maximum(li - 1, 0))


def _tw(s):
    li, lj = _li(s), _lj(s)
    return jnp.where(lj >= NA, lj - NA,
                     jnp.where(li > 0, NF - 1, 0))


def _fused_body(x0_ref, Wv_ref, Wo_ref, W1_ref, W2_ref, W3_ref,
                ln1_ref, ln2_ref, lno_ref, tab_ref, out_ref, x_s, h_s):
    s = pl.program_id(0)
    lj = s % NJ
    in_layers = s < S_HEAD

    @pl.when(s == 0)
    def _():
        x_s[...] = x0_ref[...]

    @pl.when(jnp.logical_and(in_layers, lj == 0))
    def _():
        h_s[...] = _rms(x_s[...], ln1_ref[0])

    @pl.when(jnp.logical_and(in_layers, lj == NA))
    def _():
        h_s[...] = _rms(x_s[...], ln2_ref[0])

    h = h_s[...]

    @pl.when(jnp.logical_and(in_layers, lj < NA))
    def _():
        hb = h.astype(jnp.bfloat16)
        t = jnp.dot(hb, Wv_ref[0].astype(jnp.bfloat16),
                    preferred_element_type=jnp.float32)
        x_s[...] = x_s[...] + jnp.dot(t.astype(jnp.bfloat16),
                                      Wo_ref[0].astype(jnp.bfloat16),
                                      preferred_element_type=jnp.float32)

    @pl.when(jnp.logical_and(in_layers, lj >= NA))
    def _():
        hb = h.astype(jnp.bfloat16)
        a = jnp.dot(hb, W1_ref[0].astype(jnp.bfloat16),
                    preferred_element_type=jnp.float32)
        b = jnp.dot(hb, W2_ref[0].astype(jnp.bfloat16),
                    preferred_element_type=jnp.float32)
        g = (a * jax.lax.logistic(a)) * b
        x_s[...] = x_s[...] + jnp.dot(g.astype(jnp.bfloat16),
                                      W3_ref[0].astype(jnp.bfloat16),
                                      preferred_element_type=jnp.float32)

    @pl.when(s == S_HEAD)
    def _():
        h_s[...] = _rms(x_s[...], lno_ref[...])

    @pl.when(jnp.logical_not(in_layers))
    def _():
        out_ref[...] = jax.lax.dot_general(
            h_s[...], tab_ref[...], (((1,), (1,)), ((), ())),
            preferred_element_type=jnp.float32)[:, None, :]


def _run_fused(x0, Wv, Wo, W1, W2, W3, ln1, ln2, lno, table):
    B = x0.shape[0]
    V = table.shape[0]
    nv = pl.cdiv(V, V_T)
    return pl.pallas_call(
        _fused_body,
        grid=(S_HEAD + nv,),
        in_specs=[
            pl.BlockSpec((B, D), lambda s: (0, 0)),
            pl.BlockSpec((1, D, ATT_T),
                         lambda s: (_li(s), 0, jnp.minimum(_lj(s), NA - 1))),
            pl.BlockSpec((1, ATT_T, D),
                         lambda s: (_li(s), jnp.minimum(_lj(s), NA - 1), 0)),
            pl.BlockSpec((1, D, FF_T),
                         lambda s: (_lw(s), 0, _tw(s))),
            pl.BlockSpec((1, D, FF_T),
                         lambda s: (_lw(s), 0, _tw(s))),
            pl.BlockSpec((1, FF_T, D),
                         lambda s: (_lw(s), _tw(s), 0)),
            pl.BlockSpec((1, 1, D), lambda s: (_li(s), 0, 0)),
            pl.BlockSpec((1, 1, D), lambda s: (_li(s), 0, 0)),
            pl.BlockSpec((1, D), lambda s: (0, 0)),
            pl.BlockSpec((V_T, D), lambda s: (_vt(s, nv), 0)),
        ],
        out_specs=pl.BlockSpec((B, 1, V_T), lambda s: (0, 0, _vt(s, nv))),
        out_shape=jax.ShapeDtypeStruct((B, 1, V), jnp.float32),
        scratch_shapes=[pltpu.VMEM((B, D), jnp.float32),
                        pltpu.VMEM((B, D), jnp.float32)],
    )(x0, Wv, Wo, W1, W2, W3,
      ln1.reshape(L, 1, D), ln2.reshape(L, 1, D), lno, table)


def kernel(table, Wq, Wk, Wv, Wo, W1, W2, W3, ln1, ln2, ln_out, tokens):
    B, T = tokens.shape
    assert T == 1, "kernel exploits T == 1 (single-position decode)"
    idx = tokens.reshape(1, B * T).astype(jnp.int32)
    x0 = _sc_gather(table, idx)
    return _run_fused(x0, Wv, Wo, W1, W2, W3, ln1, ln2,
                      ln_out.reshape(1, D), table)


# speedup vs baseline: 1.0185x; 1.0033x over previous
import jax
import jax.numpy as jnp
from jax.experimental import pallas as pl
from jax.experimental.pallas import tpu as pltpu
from jax.experimental.pallas import tpu_sc as plsc

D = 1024
L = 12
FF = 2816
EPS = 1e-5

ATT_T = 512
NA = D // ATT_T
FF_T = 1408
NF = FF // FF_T
NJ = NA + NF
S_HEAD = L * NJ

V_T = 2048


def _rms(x, w):
    return x * jax.lax.rsqrt(jnp.mean(x * x, axis=-1, keepdims=True) + EPS) * w


def _sc_gather(table, idx2d):
    n = idx2d.shape[1]
    mesh = plsc.VectorSubcoreMesh(core_axis_name="c", subcore_axis_name="s")

    @pl.kernel(out_type=jax.ShapeDtypeStruct((n, table.shape[1]), table.dtype),
               mesh=mesh)
    def gk(tab_hbm, i_hbm, o_hbm):
        def body(i_vmem, o_vmem):
            pltpu.sync_copy(tab_hbm.at[i_vmem.at[0]], o_vmem)

        pltpu.emit_pipeline(
            body,
            grid=(1,),
            in_specs=[pl.BlockSpec((1, n), lambda i: (0, 0))],
            out_specs=[pl.BlockSpec((n, table.shape[1]), lambda i: (0, 0))],
            core_axis_name="s",
            dimension_semantics=(pltpu.PARALLEL,),
        )(i_hbm, o_hbm)

    return gk(table, idx2d)


def _li(s):
    return jnp.minimum(s // NJ, L - 1)


def _lj(s):
    return jnp.where(s < S_HEAD, s % NJ, NJ - 1)


def _vt(s, nv):
    return jnp.clip(s - S_HEAD, 0, nv - 1)


def _lw(s):
    li, lj = _li(s), _lj(s)
    return jnp.where(lj >= NA, li, jnp.maximum(li - 1, 0))


def _tw(s):
    li, lj = _li(s), _lj(s)
    return jnp.where(lj >= NA, lj - NA,
                     jnp.where(li > 0, NF - 1, 0))


def _fused_body(x0_ref, Wv_ref, Wo_ref, W1_ref, W2_ref, W3_ref,
                ln1_ref, ln2_ref, lno_ref, tab_ref, out_ref, x_s, h_s):
    s = pl.program_id(0)
    lj = s % NJ
    in_layers = s < S_HEAD

    @pl.when(s == 0)
    def _():
        x_s[...] = x0_ref[...]

    @pl.when(jnp.logical_and(in_layers, lj == 0))
    def _():
        h_s[...] = _rms(x_s[...], ln1_ref[0])

    @pl.when(jnp.logical_and(in_layers, lj == NA))
    def _():
        h_s[...] = _rms(x_s[...], ln2_ref[0])

    h = h_s[...]

    @pl.when(jnp.logical_and(in_layers, lj < NA))
    def _():
        t = jnp.dot(h, Wv_ref[0], preferred_element_type=jnp.float32)
        x_s[...] = x_s[...] + jnp.dot(t, Wo_ref[0],
                                      preferred_element_type=jnp.float32)

    @pl.when(jnp.logical_and(in_layers, lj >= NA))
    def _():
        a = jnp.dot(h, W1_ref[0], preferred_element_type=jnp.float32)
        b = jnp.dot(h, W2_ref[0], preferred_element_type=jnp.float32)
        g = (a * jax.lax.logistic(a)) * b
        x_s[...] = x_s[...] + jnp.dot(g, W3_ref[0],
                                      preferred_element_type=jnp.float32)

    @pl.when(s == S_HEAD)
    def _():
        h_s[...] = _rms(x_s[...], lno_ref[...])

    @pl.when(jnp.logical_not(in_layers))
    def _():
        out_ref[...] = jax.lax.dot_general(
            h_s[...], tab_ref[...], (((1,), (1,)), ((), ())),
            preferred_element_type=jnp.float32)[:, None, :]


def _run_fused(x0, Wv, Wo, W1, W2, W3, ln1, ln2, lno, table):
    B = x0.shape[0]
    V = table.shape[0]
    nv = pl.cdiv(V, V_T)
    return pl.pallas_call(
        _fused_body,
        grid=(S_HEAD + nv,),
        in_specs=[
            pl.BlockSpec((B, D), lambda s: (0, 0)),
            pl.BlockSpec((1, D, ATT_T),
                         lambda s: (_li(s), 0, jnp.minimum(_lj(s), NA - 1))),
            pl.BlockSpec((1, ATT_T, D),
                         lambda s: (_li(s), jnp.minimum(_lj(s), NA - 1), 0)),
            pl.BlockSpec((1, D, FF_T),
                         lambda s: (_lw(s), 0, _tw(s))),
            pl.BlockSpec((1, D, FF_T),
                         lambda s: (_lw(s), 0, _tw(s))),
            pl.BlockSpec((1, FF_T, D),
                         lambda s: (_lw(s), _tw(s), 0)),
            pl.BlockSpec((1, 1, D), lambda s: (_li(s), 0, 0)),
            pl.BlockSpec((1, 1, D), lambda s: (_li(s), 0, 0)),
            pl.BlockSpec((1, D), lambda s: (0, 0)),
            pl.BlockSpec((V_T, D), lambda s: (_vt(s, nv), 0)),
        ],
        out_specs=pl.BlockSpec((B, 1, V_T), lambda s: (0, 0, _vt(s, nv))),
        out_shape=jax.ShapeDtypeStruct((B, 1, V), jnp.float32),
        scratch_shapes=[pltpu.VMEM((B, D), jnp.float32),
                        pltpu.VMEM((B, D), jnp.float32)],
    )(x0, Wv, Wo, W1, W2, W3,
      ln1.reshape(L, 1, D), ln2.reshape(L, 1, D), lno, table)


def kernel(table, Wq, Wk, Wv, Wo, W1, W2, W3, ln1, ln2, ln_out, tokens):
    B, T = tokens.shape
    assert T == 1, "kernel exploits T == 1 (single-position decode)"
    idx = tokens.reshape(1, B * T).astype(jnp.int32)
    x0 = _sc_gather(table, idx)
    return _run_fused(x0, Wv, Wo, W1, W2, W3, ln1, ln2,
                      ln_out.reshape(1, D), table)
